# baseline (speedup 1.0000x reference)
import functools

import jax
import jax.numpy as jnp
from jax import lax
from jax.experimental import pallas as pl
from jax.experimental.pallas import tpu as pltpu
from jax.experimental.pallas import tpu_sc as plsc

_NW = 32
_L = 16


@functools.partial(jax.jit, static_argnames=("n_chunks",))
def _sc_gather(x, table, *, n_chunks):
    b, h = x.shape
    d = table.shape[1]
    b_per_w = b // _NW
    n_per_w = b_per_w * h
    chunk = n_per_w // n_chunks
    n128 = b * h * d // 128
    pack = 128 // d
    mesh = plsc.VectorSubcoreMesh(core_axis_name="c", subcore_axis_name="s")

    @functools.partial(
        pl.kernel,
        mesh=mesh,
        out_type=jax.ShapeDtypeStruct((n128, 128), jnp.float32),
        scratch_types=[
            pltpu.VMEM((b_per_w, h), jnp.int32),
            pltpu.VMEM((n_per_w,), jnp.int32),
            pltpu.VMEM((chunk, d), jnp.float32),
            pltpu.VMEM((chunk // pack, 128), jnp.float32),
            pltpu.SemaphoreType.DMA,
        ],
        compiler_params=pltpu.CompilerParams(use_tc_tiling_on_sc=False),
    )
    def k(x_hbm, table_hbm, out_hbm, idx2d_v, idx_v, rows_v, r128_v, sem):
        wid = lax.axis_index("s") * 2 + lax.axis_index("c")
        base_b = wid * b_per_w
        base_n = wid * n_per_w
        pltpu.sync_copy(x_hbm.at[pl.ds(base_b, b_per_w), :], idx2d_v)

        col_offs = [c * _L for c in range(h // _L)] + [h - _L]

        def flatten_step(r, _):
            for c in col_offs:
                idx_v[pl.ds(r * h + c, _L)] = idx2d_v[r, pl.ds(c, _L)]
            return 0

        lax.fori_loop(0, b_per_w, flatten_step, 0)

        def repack_step(j, _):
            for l in range(128 // _L):
                r128_v[j, pl.ds(l * _L, _L)] = (
                    rows_v[pack * j + l * _L // d, pl.ds(l * _L % d, _L)])
            return 0

        def chunk_step(ch, _):
            idx_ref = idx_v.at[pl.ds(ch * chunk, chunk)]
            pltpu.async_copy(table_hbm.at[idx_ref], rows_v, sem).wait()
            lax.fori_loop(0, chunk // pack, repack_step, 0)
            pltpu.sync_copy(
                r128_v,
                out_hbm.at[pl.ds((base_n + ch * chunk) // pack, chunk // pack),
                           :])
            return 0

        lax.fori_loop(0, n_chunks, chunk_step, 0)

    return k(x, table)


def _tc_expand(o128, b, h, d):
    bb = 128
    rows_b = bb * h * d // 128

    def body(i_ref, o_ref, s_ref):
        x = i_ref[...]
        quarters = [x[:, q * d:(q + 1) * d] for q in range(128 // d)]
        s_ref[...] = jnp.stack(quarters, axis=1).reshape(bb * h, d)
        o_ref[...] = s_ref[...].reshape(bb, h, d)

    return pl.pallas_call(
        body,
        grid=(b // bb,),
        in_specs=[pl.BlockSpec((rows_b, 128), lambda i: (i, 0))],
        out_specs=pl.BlockSpec((bb, h, d), lambda i: (i, 0, 0)),
        out_shape=jax.ShapeDtypeStruct((b, h, d), jnp.float32),
        scratch_shapes=[pltpu.VMEM((bb * h, d), jnp.float32)],
    )(o128)


def kernel(x, table):
    b, h = x.shape
    d = table.shape[1]
    o128 = _sc_gather(x, table, n_chunks=32)
    return _tc_expand(o128, b, h, d)


# speedup vs baseline: 1.6588x; 1.6588x over previous
import functools

import jax
import jax.numpy as jnp
from jax import lax
from jax.experimental import pallas as pl
from jax.experimental.pallas import tpu as pltpu
from jax.experimental.pallas import tpu_sc as plsc

_NW = 32
_L = 16


@functools.partial(jax.jit, static_argnames=("n_chunks",))
def _sc_gather(x, table, *, n_chunks):
    b, h = x.shape
    d = table.shape[1]
    b_per_w = b // _NW
    b_chunk = b_per_w // n_chunks
    n_per_w = b_per_w * h
    chunk = b_chunk * h
    mesh = plsc.VectorSubcoreMesh(core_axis_name="c", subcore_axis_name="s")

    @functools.partial(
        pl.kernel,
        mesh=mesh,
        out_type=jax.ShapeDtypeStruct((b, h, d), jnp.float32),
        scratch_types=[
            pltpu.VMEM((b_per_w, h), jnp.int32),
            pltpu.VMEM((n_per_w,), jnp.int32),
            pltpu.VMEM((2, chunk, d), jnp.float32),
            pltpu.SemaphoreType.DMA,
            pltpu.SemaphoreType.DMA,
            pltpu.SemaphoreType.DMA,
        ],
        compiler_params=pltpu.CompilerParams(use_tc_tiling_on_sc=False),
    )
    def k(x_hbm, table_hbm, out_hbm, idx2d_v, idx_v, rows_v, sem0, sem1,
          sem_s):
        sems = (sem0, sem1)
        wid = lax.axis_index("s") * 2 + lax.axis_index("c")
        base_b = wid * b_per_w
        pltpu.sync_copy(x_hbm.at[pl.ds(base_b, b_per_w), :], idx2d_v)

        col_offs = [c * _L for c in range(h // _L)] + [h - _L]

        def flatten_step(r, _):
            for c in col_offs:
                idx_v[pl.ds(r * h + c, _L)] = idx2d_v[r, pl.ds(c, _L)]
            return 0

        lax.fori_loop(0, b_per_w, flatten_step, 0)

        def start_gather(ch, s):
            pltpu.async_copy(table_hbm.at[idx_v.at[pl.ds(ch * chunk, chunk)]],
                             rows_v.at[s], sems[s])

        def wait_gather(s):
            pltpu.make_async_copy(
                table_hbm.at[idx_v.at[pl.ds(0, chunk)]],
                rows_v.at[s], sems[s]).wait()

        start_gather(0, 0)
        start_gather(1, 1)

        def pair_body(g, _):
            for s in (0, 1):
                ch = 2 * g + s
                wait_gather(s)
                bb0 = base_b + ch * b_chunk

                def fire(bb, _):
                    pltpu.async_copy(rows_v.at[s, pl.ds(bb * h, h), :],
                                     out_hbm.at[bb0 + bb], sem_s)
                    return 0

                lax.fori_loop(0, b_chunk, fire, 0)

                def drain(bb, _):
                    pltpu.make_async_copy(rows_v.at[s, pl.ds(0, h), :],
                                          out_hbm.at[base_b], sem_s).wait()
                    return 0

                lax.fori_loop(0, b_chunk, drain, 0)

                @pl.when(ch + 2 < n_chunks)
                def _():
                    start_gather(ch + 2, s)
            return 0

        lax.fori_loop(0, n_chunks // 2, pair_body, 0)

    return k(x, table)


def kernel(x, table):
    return _sc_gather(x, table, n_chunks=32)
